# baseline (speedup 1.0000x reference)
import functools

import jax
import jax.numpy as jnp
from jax import lax
from jax.experimental import pallas as pl
from jax.experimental.pallas import tpu as pltpu
from jax.experimental.pallas import tpu_sc as plsc

D_MODEL = 64
SCALE = 8.0
NC, NS = 2, 16
NW = NC * NS
VOCAB = 1000000
PACK = VOCAB // 2
BATCH = 4096
SEQ = 200
LANES = 16
BLK = 128
NBLK_FULL = 7812
PER_W1 = NBLK_FULL // NW


def _iota16():
    return lax.iota(jnp.int32, LANES)


def _relayout_body(tab_hbm, dense_hbm, inb0, inb1, outb0, outb1, inp,
                   si0, si1, so0, so1):
    c = lax.axis_index("c")
    s = lax.axis_index("s")
    wid = s * NC + c
    inbs, outbs = (inb0, inb1), (outb0, outb1)
    sis, sos = (si0, si1), (so0, so1)
    it16 = _iota16()
    rvecs = [it16 + 16 * (g % 4) for g in range(8)]

    def in_src(b):
        return tab_hbm.at[:, pl.ds(b * BLK, BLK)]

    def issue_in(b, r):
        pltpu.async_copy(in_src(b), inbs[r], sis[r])

    def wait_in(b, r):
        pltpu.make_async_copy(in_src(b), inbs[r], sis[r]).wait()

    def out_dst(b):
        return dense_hbm.at[pl.ds(b * 64, 64)]

    def issue_out(b, r):
        pltpu.async_copy(outbs[r], out_dst(b), sos[r])

    def wait_out(b, r):
        pltpu.make_async_copy(outbs[r], out_dst(b), sos[r]).wait()

    def transpose_block(r):
        inb, outb = inbs[r], outbs[r]

        def rowp(p, _):
            c0 = jnp.broadcast_to(2 * p, (LANES,)).astype(jnp.int32)
            c1 = c0 + 1
            for g in range(8):
                cv = c0 if g < 4 else c1
                v = plsc.load_gather(inb, [rvecs[g], cv]) * SCALE
                outb[p, pl.ds(16 * g, LANES)] = v
            return 0

        lax.fori_loop(0, 64, rowp, 0)

    issue_in(wid, 0)

    def pair_body(m, _):
        n0 = 2 * m
        b0 = wid + NW * n0
        wait_in(b0, 0)
        issue_in(b0 + NW, 1)

        @pl.when(m >= 1)
        def _():
            wait_out(b0 - 2 * NW, 0)

        transpose_block(0)
        issue_out(b0, 0)
        b1 = b0 + NW
        wait_in(b1, 1)

        @pl.when(m + 1 < PER_W1 // 2)
        def _():
            issue_in(b1 + NW, 0)

        @pl.when(m >= 1)
        def _():
            wait_out(b1 - 2 * NW, 1)

        transpose_block(1)
        issue_out(b1, 1)
        return 0

    lax.fori_loop(0, PER_W1 // 2, pair_body, 0)
    wait_out(wid + NW * (PER_W1 - 2), 0)
    wait_out(wid + NW * (PER_W1 - 1), 1)

    @pl.when(wid < 4)
    def _():
        b = NBLK_FULL - 4 + wid
        pltpu.async_copy(in_src(b), inbs[0], sis[0])
        pltpu.make_async_copy(in_src(b), inbs[0], sis[0]).wait()
        transpose_block(0)
        pltpu.async_copy(outbs[0], out_dst(b), sos[0])
        pltpu.make_async_copy(outbs[0], out_dst(b), sos[0]).wait()

    @pl.when(wid == 4)
    def _():
        src = tab_hbm.at[:, pl.ds(NBLK_FULL * BLK, 64)]
        dst = inp
        pltpu.async_copy(src, dst, sis[0])
        pltpu.make_async_copy(src, dst, sis[0]).wait()
        it = _iota16()

        def rowp(p, _):
            c0 = jnp.broadcast_to(2 * p, (LANES,)).astype(jnp.int32)
            c1 = c0 + 1
            for g in range(8):
                cv = c0 if g < 4 else c1
                rv = it + 16 * (g % 4)
                v = plsc.load_gather(inp, [rv, cv]) * SCALE
                outbs[0][p, pl.ds(16 * g, LANES)] = v
            return 0

        lax.fori_loop(0, 32, rowp, 0)
        odst = dense_hbm.at[pl.ds(NBLK_FULL * 64, 32)]
        pltpu.async_copy(outbs[0].at[pl.ds(0, 32)], odst, sos[0])
        pltpu.make_async_copy(outbs[0].at[pl.ds(0, 32)], odst, sos[0]).wait()


_relayout = functools.partial(
    pl.kernel,
    out_type=jax.ShapeDtypeStruct((PACK, BLK), jnp.float32),
    mesh=plsc.VectorSubcoreMesh(core_axis_name="c", subcore_axis_name="s"),
    compiler_params=pltpu.CompilerParams(use_tc_tiling_on_sc=True, needs_layout_passes=False),
    scratch_types=(
        [pltpu.VMEM((D_MODEL, BLK), jnp.float32) for _ in range(2)]
        + [pltpu.VMEM((D_MODEL, BLK), jnp.float32) for _ in range(2)]
        + [pltpu.VMEM((D_MODEL, 64), jnp.float32)]
        + [pltpu.SemaphoreType.DMA for _ in range(4)]
    ),
)(_relayout_body)


NBUF2 = 3


def _lookup_body(xt_hbm, dense_hbm, out_hbm, idx_v, pb0, pb1, pb2,
                 gb0, gb1, gb2, ob0, ob1, sg0, sg1, sg2, so0, so1):
    c = lax.axis_index("c")
    s = lax.axis_index("s")
    wid = s * NC + c
    i0 = wid * BLK
    pbs, gbs = (pb0, pb1, pb2), (gb0, gb1, gb2)
    obs = (ob0, ob1)
    sgs, sos = (sg0, sg1, sg2), (so0, so1)
    it16 = _iota16()
    kvecs = [it16 + 16 * g for g in range(8)]

    pltpu.sync_copy(xt_hbm.at[:, pl.ds(i0, BLK)], idx_v)

    def prep(t, r):
        pb = pbs[r]
        for g in range(8):
            sl = pl.ds(16 * g, LANES)
            pb[sl] = lax.shift_right_logical(idx_v[t, sl], 1)

    def issue_gather(t, r):
        pltpu.async_copy(dense_hbm.at[pbs[r]], gbs[r], sgs[r])

    def wait_gather(t, r):
        pltpu.make_async_copy(dense_hbm.at[pbs[r]], gbs[r], sgs[r]).wait()

    def out_dst(t):
        return out_hbm.at[t, :, pl.ds(i0, BLK)]

    def issue_out(t, r):
        pltpu.async_copy(obs[r], out_dst(t), sos[r])

    def wait_out(t, r):
        pltpu.make_async_copy(obs[r], out_dst(t), sos[r]).wait()

    def transpose_block(t, rg, ro):
        gb, ob = gbs[rg], obs[ro]
        hvs = [(idx_v[t, pl.ds(16 * g, LANES)] & 1) * D_MODEL
               for g in range(8)]

        def rowd(d, _):
            for g in range(8):
                v = plsc.load_gather(gb, [kvecs[g], hvs[g] + d])
                ob[d, pl.ds(16 * g, LANES)] = v
            return 0

        lax.fori_loop(0, D_MODEL, rowd, 0)

    for t in range(NBUF2):
        prep(t, t)
        issue_gather(t, t)

    def six_body(m, _):
        base = 6 * m
        for j in range(6):
            t = base + j
            rg = j % NBUF2
            ro = j % 2
            wait_gather(t, rg)

            @pl.when(t >= 2)
            def _():
                wait_out(t - 2, ro)

            transpose_block(t, rg, ro)
            issue_out(t, ro)

            @pl.when(t + NBUF2 < SEQ)
            def _():
                prep(t + NBUF2, rg)
                issue_gather(t + NBUF2, rg)
        return 0

    lax.fori_loop(0, 33, six_body, 0)
    for t in (198, 199):
        rg = t % 3
        ro = t % 2
        wait_gather(t, rg)
        wait_out(t - 2, ro)
        transpose_block(t, rg, ro)
        issue_out(t, ro)
    wait_out(198, 0)
    wait_out(199, 1)


_lookup = functools.partial(
    pl.kernel,
    out_type=jax.ShapeDtypeStruct((SEQ, D_MODEL, BATCH), jnp.float32),
    mesh=plsc.VectorSubcoreMesh(core_axis_name="c", subcore_axis_name="s"),
    compiler_params=pltpu.CompilerParams(use_tc_tiling_on_sc=True, needs_layout_passes=False),
    scratch_types=(
        [pltpu.VMEM((SEQ, BLK), jnp.int32)]
        + [pltpu.VMEM((BLK,), jnp.int32) for _ in range(NBUF2)]
        + [pltpu.VMEM((BLK, BLK), jnp.float32) for _ in range(NBUF2)]
        + [pltpu.VMEM((D_MODEL, BLK), jnp.float32) for _ in range(2)]
        + [pltpu.SemaphoreType.DMA for _ in range(NBUF2 + 2)]
    ),
)(_lookup_body)


@jax.jit
def kernel(x, table):
    dense = _relayout(table.T)
    out = _lookup(x.T, dense)
    return out.transpose(2, 0, 1)


# speedup vs baseline: 1.5021x; 1.5021x over previous
import functools

import jax
import jax.numpy as jnp
from jax import lax
from jax.experimental import pallas as pl
from jax.experimental.pallas import tpu as pltpu
from jax.experimental.pallas import tpu_sc as plsc

D_MODEL = 64
SCALE = 8.0
NC, NS = 2, 16
NW = NC * NS
VOCAB = 1000000
PACK = VOCAB // 2
BATCH = 4096
SEQ = 200
LANES = 16
BLK = 128
NBLK_FULL = 7812
PER_W1 = NBLK_FULL // NW


def _iota16():
    return lax.iota(jnp.int32, LANES)


def _relayout_body(tab_hbm, dense_hbm, inb0, inb1, outb0, outb1, inp,
                   si0, si1, so0, so1):
    c = lax.axis_index("c")
    s = lax.axis_index("s")
    wid = s * NC + c
    inbs, outbs = (inb0, inb1), (outb0, outb1)
    sis, sos = (si0, si1), (so0, so1)
    it16 = _iota16()
    rvecs = [it16 + 16 * (g % 4) for g in range(8)]

    def in_src(b):
        return tab_hbm.at[:, pl.ds(b * BLK, BLK)]

    def issue_in(b, r):
        pltpu.async_copy(in_src(b), inbs[r], sis[r])

    def wait_in(b, r):
        pltpu.make_async_copy(in_src(b), inbs[r], sis[r]).wait()

    def out_dst(b):
        return dense_hbm.at[pl.ds(b * 64, 64)]

    def issue_out(b, r):
        pltpu.async_copy(outbs[r], out_dst(b), sos[r])

    def wait_out(b, r):
        pltpu.make_async_copy(outbs[r], out_dst(b), sos[r]).wait()

    def transpose_block(r):
        inb, outb = inbs[r], outbs[r]

        @plsc.parallel_loop(0, 64, unroll=2)
        def rowp(p):
            c0 = jnp.broadcast_to(2 * p, (LANES,)).astype(jnp.int32)
            c1 = c0 + 1
            vs = [plsc.load_gather(inb, [rvecs[g], c0 if g < 4 else c1])
                  for g in range(8)]
            for g in range(8):
                outb[p, pl.ds(16 * g, LANES)] = vs[g] * SCALE

    issue_in(wid, 0)

    def pair_body(m, _):
        n0 = 2 * m
        b0 = wid + NW * n0
        wait_in(b0, 0)
        issue_in(b0 + NW, 1)

        @pl.when(m >= 1)
        def _():
            wait_out(b0 - 2 * NW, 0)

        transpose_block(0)
        issue_out(b0, 0)
        b1 = b0 + NW
        wait_in(b1, 1)

        @pl.when(m + 1 < PER_W1 // 2)
        def _():
            issue_in(b1 + NW, 0)

        @pl.when(m >= 1)
        def _():
            wait_out(b1 - 2 * NW, 1)

        transpose_block(1)
        issue_out(b1, 1)
        return 0

    lax.fori_loop(0, PER_W1 // 2, pair_body, 0)
    wait_out(wid + NW * (PER_W1 - 2), 0)
    wait_out(wid + NW * (PER_W1 - 1), 1)

    @pl.when(wid < 4)
    def _():
        b = NBLK_FULL - 4 + wid
        pltpu.async_copy(in_src(b), inbs[0], sis[0])
        pltpu.make_async_copy(in_src(b), inbs[0], sis[0]).wait()
        transpose_block(0)
        pltpu.async_copy(outbs[0], out_dst(b), sos[0])
        pltpu.make_async_copy(outbs[0], out_dst(b), sos[0]).wait()

    @pl.when(wid == 4)
    def _():
        src = tab_hbm.at[:, pl.ds(NBLK_FULL * BLK, 64)]
        dst = inp
        pltpu.async_copy(src, dst, sis[0])
        pltpu.make_async_copy(src, dst, sis[0]).wait()
        it = _iota16()

        @plsc.parallel_loop(0, 32, unroll=2)
        def rowp(p):
            c0 = jnp.broadcast_to(2 * p, (LANES,)).astype(jnp.int32)
            c1 = c0 + 1
            vs = [plsc.load_gather(inp, [it + 16 * (g % 4),
                                         c0 if g < 4 else c1])
                  for g in range(8)]
            for g in range(8):
                outbs[0][p, pl.ds(16 * g, LANES)] = vs[g] * SCALE
        odst = dense_hbm.at[pl.ds(NBLK_FULL * 64, 32)]
        pltpu.async_copy(outbs[0].at[pl.ds(0, 32)], odst, sos[0])
        pltpu.make_async_copy(outbs[0].at[pl.ds(0, 32)], odst, sos[0]).wait()


_relayout = functools.partial(
    pl.kernel,
    out_type=jax.ShapeDtypeStruct((PACK, BLK), jnp.float32),
    mesh=plsc.VectorSubcoreMesh(core_axis_name="c", subcore_axis_name="s"),
    compiler_params=pltpu.CompilerParams(use_tc_tiling_on_sc=True, needs_layout_passes=False),
    scratch_types=(
        [pltpu.VMEM((D_MODEL, BLK), jnp.float32) for _ in range(2)]
        + [pltpu.VMEM((D_MODEL, BLK), jnp.float32) for _ in range(2)]
        + [pltpu.VMEM((D_MODEL, 64), jnp.float32)]
        + [pltpu.SemaphoreType.DMA for _ in range(4)]
    ),
)(_relayout_body)


NBUF2 = 3


def _lookup_body(xt_hbm, dense_hbm, out_hbm, idx_v, pb0, pb1, pb2,
                 gb0, gb1, gb2, ob0, ob1, sg0, sg1, sg2, so0, so1):
    c = lax.axis_index("c")
    s = lax.axis_index("s")
    wid = s * NC + c
    i0 = wid * BLK
    pbs, gbs = (pb0, pb1, pb2), (gb0, gb1, gb2)
    obs = (ob0, ob1)
    sgs, sos = (sg0, sg1, sg2), (so0, so1)
    it16 = _iota16()
    kvecs = [it16 + 16 * g for g in range(8)]

    pltpu.sync_copy(xt_hbm.at[:, pl.ds(i0, BLK)], idx_v)

    def prep(t, r):
        pb = pbs[r]
        for g in range(8):
            sl = pl.ds(16 * g, LANES)
            pb[sl] = lax.shift_right_logical(idx_v[t, sl], 1)

    def issue_gather(t, r):
        pltpu.async_copy(dense_hbm.at[pbs[r]], gbs[r], sgs[r])

    def wait_gather(t, r):
        pltpu.make_async_copy(dense_hbm.at[pbs[r]], gbs[r], sgs[r]).wait()

    def out_dst(t):
        return out_hbm.at[t, :, pl.ds(i0, BLK)]

    def issue_out(t, r):
        pltpu.async_copy(obs[r], out_dst(t), sos[r])

    def wait_out(t, r):
        pltpu.make_async_copy(obs[r], out_dst(t), sos[r]).wait()

    def transpose_block(t, rg, ro):
        gb, ob = gbs[rg], obs[ro]
        hvs = [(idx_v[t, pl.ds(16 * g, LANES)] & 1) * D_MODEL
               for g in range(8)]

        @plsc.parallel_loop(0, D_MODEL, unroll=2)
        def rowd(d):
            vs = [plsc.load_gather(gb, [kvecs[g], hvs[g] + d])
                  for g in range(8)]
            for g in range(8):
                ob[d, pl.ds(16 * g, LANES)] = vs[g]

    for t in range(NBUF2):
        prep(t, t)
        issue_gather(t, t)

    def six_body(m, _):
        base = 6 * m
        for j in range(6):
            t = base + j
            rg = j % NBUF2
            ro = j % 2
            wait_gather(t, rg)

            @pl.when(t >= 2)
            def _():
                wait_out(t - 2, ro)

            transpose_block(t, rg, ro)
            issue_out(t, ro)

            @pl.when(t + NBUF2 < SEQ)
            def _():
                prep(t + NBUF2, rg)
                issue_gather(t + NBUF2, rg)
        return 0

    lax.fori_loop(0, 33, six_body, 0)
    for t in (198, 199):
        rg = t % 3
        ro = t % 2
        wait_gather(t, rg)
        wait_out(t - 2, ro)
        transpose_block(t, rg, ro)
        issue_out(t, ro)
    wait_out(198, 0)
    wait_out(199, 1)


_lookup = functools.partial(
    pl.kernel,
    out_type=jax.ShapeDtypeStruct((SEQ, D_MODEL, BATCH), jnp.float32),
    mesh=plsc.VectorSubcoreMesh(core_axis_name="c", subcore_axis_name="s"),
    compiler_params=pltpu.CompilerParams(use_tc_tiling_on_sc=True, needs_layout_passes=False),
    scratch_types=(
        [pltpu.VMEM((SEQ, BLK), jnp.int32)]
        + [pltpu.VMEM((BLK,), jnp.int32) for _ in range(NBUF2)]
        + [pltpu.VMEM((BLK, BLK), jnp.float32) for _ in range(NBUF2)]
        + [pltpu.VMEM((D_MODEL, BLK), jnp.float32) for _ in range(2)]
        + [pltpu.SemaphoreType.DMA for _ in range(NBUF2 + 2)]
    ),
)(_lookup_body)


@jax.jit
def kernel(x, table):
    dense = _relayout(table.T)
    out = _lookup(x.T, dense)
    return out.transpose(2, 0, 1)


# speedup vs baseline: 1.5057x; 1.0024x over previous
import functools

import jax
import jax.numpy as jnp
from jax import lax
from jax.experimental import pallas as pl
from jax.experimental.pallas import tpu as pltpu
from jax.experimental.pallas import tpu_sc as plsc

D_MODEL = 64
SCALE = 8.0
NC, NS = 2, 16
NW = NC * NS
VOCAB = 1000000
PACK = VOCAB // 2
BATCH = 4096
SEQ = 200
LANES = 16
BLK = 128
NBLK_FULL = 7812
PER_W1 = NBLK_FULL // NW


def _iota16():
    return lax.iota(jnp.int32, LANES)


def _relayout_body(tab_hbm, dense_hbm, inb0, inb1, outb0, outb1, inp,
                   si0, si1, so0, so1):
    c = lax.axis_index("c")
    s = lax.axis_index("s")
    wid = s * NC + c
    inbs, outbs = (inb0, inb1), (outb0, outb1)
    sis, sos = (si0, si1), (so0, so1)
    it16 = _iota16()
    rvecs = [it16 + 16 * (g % 4) for g in range(8)]

    def in_src(b):
        return tab_hbm.at[:, pl.ds(b * BLK, BLK)]

    def issue_in(b, r):
        pltpu.async_copy(in_src(b), inbs[r], sis[r])

    def wait_in(b, r):
        pltpu.make_async_copy(in_src(b), inbs[r], sis[r]).wait()

    def out_dst(b):
        return dense_hbm.at[pl.ds(b * 64, 64)]

    def issue_out(b, r):
        pltpu.async_copy(outbs[r], out_dst(b), sos[r])

    def wait_out(b, r):
        pltpu.make_async_copy(outbs[r], out_dst(b), sos[r]).wait()

    def transpose_block(r):
        inb, outb = inbs[r], outbs[r]

        @plsc.parallel_loop(0, 64, unroll=2)
        def rowp(p):
            c0 = jnp.broadcast_to(2 * p, (LANES,)).astype(jnp.int32)
            c1 = c0 + 1
            vs = [plsc.load_gather(inb, [rvecs[g], c0 if g < 4 else c1])
                  for g in range(8)]
            for g in range(8):
                outb[p, pl.ds(16 * g, LANES)] = vs[g] * SCALE

    issue_in(wid, 0)

    def pair_body(m, _):
        n0 = 2 * m
        b0 = wid + NW * n0
        wait_in(b0, 0)
        issue_in(b0 + NW, 1)

        @pl.when(m >= 1)
        def _():
            wait_out(b0 - 2 * NW, 0)

        transpose_block(0)
        issue_out(b0, 0)
        b1 = b0 + NW
        wait_in(b1, 1)

        @pl.when(m + 1 < PER_W1 // 2)
        def _():
            issue_in(b1 + NW, 0)

        @pl.when(m >= 1)
        def _():
            wait_out(b1 - 2 * NW, 1)

        transpose_block(1)
        issue_out(b1, 1)
        return 0

    lax.fori_loop(0, PER_W1 // 2, pair_body, 0)
    wait_out(wid + NW * (PER_W1 - 2), 0)
    wait_out(wid + NW * (PER_W1 - 1), 1)

    @pl.when(wid < 4)
    def _():
        b = NBLK_FULL - 4 + wid
        pltpu.async_copy(in_src(b), inbs[0], sis[0])
        pltpu.make_async_copy(in_src(b), inbs[0], sis[0]).wait()
        transpose_block(0)
        pltpu.async_copy(outbs[0], out_dst(b), sos[0])
        pltpu.make_async_copy(outbs[0], out_dst(b), sos[0]).wait()

    @pl.when(wid == 4)
    def _():
        src = tab_hbm.at[:, pl.ds(NBLK_FULL * BLK, 64)]
        dst = inp
        pltpu.async_copy(src, dst, sis[0])
        pltpu.make_async_copy(src, dst, sis[0]).wait()
        it = _iota16()

        @plsc.parallel_loop(0, 32, unroll=2)
        def rowp(p):
            c0 = jnp.broadcast_to(2 * p, (LANES,)).astype(jnp.int32)
            c1 = c0 + 1
            vs = [plsc.load_gather(inp, [it + 16 * (g % 4),
                                         c0 if g < 4 else c1])
                  for g in range(8)]
            for g in range(8):
                outbs[0][p, pl.ds(16 * g, LANES)] = vs[g] * SCALE
        odst = dense_hbm.at[pl.ds(NBLK_FULL * 64, 32)]
        pltpu.async_copy(outbs[0].at[pl.ds(0, 32)], odst, sos[0])
        pltpu.make_async_copy(outbs[0].at[pl.ds(0, 32)], odst, sos[0]).wait()


_relayout = functools.partial(
    pl.kernel,
    out_type=jax.ShapeDtypeStruct((PACK, BLK), jnp.float32),
    mesh=plsc.VectorSubcoreMesh(core_axis_name="c", subcore_axis_name="s"),
    compiler_params=pltpu.CompilerParams(use_tc_tiling_on_sc=True, needs_layout_passes=False),
    scratch_types=(
        [pltpu.VMEM((D_MODEL, BLK), jnp.float32) for _ in range(2)]
        + [pltpu.VMEM((D_MODEL, BLK), jnp.float32) for _ in range(2)]
        + [pltpu.VMEM((D_MODEL, 64), jnp.float32)]
        + [pltpu.SemaphoreType.DMA for _ in range(4)]
    ),
)(_relayout_body)


NBUF2 = 3


def _lookup_body(xt_hbm, dense_hbm, out_hbm, idx_v, hall,
                 gb0, gb1, gb2, ob0, ob1, sg0, sg1, sg2, so0, so1):
    c = lax.axis_index("c")
    s = lax.axis_index("s")
    wid = s * NC + c
    i0 = wid * BLK
    gbs = (gb0, gb1, gb2)
    obs = (ob0, ob1)
    sgs, sos = (sg0, sg1, sg2), (so0, so1)
    it16 = _iota16()
    kvecs = [it16 + 16 * g for g in range(8)]

    pltpu.sync_copy(xt_hbm.at[:, pl.ds(i0, BLK)], idx_v)

    @plsc.parallel_loop(0, SEQ, unroll=2)
    def _prep(t):
        for g in range(8):
            sl = pl.ds(16 * g, LANES)
            v = idx_v[t, sl]
            hall[t, sl] = (v & 1) * D_MODEL
            idx_v[t, sl] = lax.shift_right_logical(v, 1)

    def issue_gather(t, r):
        pltpu.async_copy(dense_hbm.at[idx_v.at[t]], gbs[r], sgs[r])

    def wait_gather(t, r):
        pltpu.make_async_copy(dense_hbm.at[idx_v.at[t]], gbs[r], sgs[r]).wait()

    def out_dst(t):
        return out_hbm.at[t, :, pl.ds(i0, BLK)]

    def issue_out(t, r):
        pltpu.async_copy(obs[r], out_dst(t), sos[r])

    def wait_out(t, r):
        pltpu.make_async_copy(obs[r], out_dst(t), sos[r]).wait()

    def transpose_block(t, rg, ro):
        gb, ob = gbs[rg], obs[ro]
        hvs = [hall[t, pl.ds(16 * g, LANES)] for g in range(8)]

        @plsc.parallel_loop(0, D_MODEL, unroll=2)
        def rowd(d):
            vs = [plsc.load_gather(gb, [kvecs[g], hvs[g] + d])
                  for g in range(8)]
            for g in range(8):
                ob[d, pl.ds(16 * g, LANES)] = vs[g]

    for t in range(NBUF2):
        issue_gather(t, t)

    def six_body(m, _):
        base = 6 * m
        for j in range(6):
            t = base + j
            rg = j % NBUF2
            ro = j % 2
            wait_gather(t, rg)

            @pl.when(t >= 2)
            def _():
                wait_out(t - 2, ro)

            transpose_block(t, rg, ro)
            issue_out(t, ro)

            @pl.when(t + NBUF2 < SEQ)
            def _():
                issue_gather(t + NBUF2, rg)
        return 0

    lax.fori_loop(0, 33, six_body, 0)
    for t in (198, 199):
        rg = t % 3
        ro = t % 2
        wait_gather(t, rg)
        wait_out(t - 2, ro)
        transpose_block(t, rg, ro)
        issue_out(t, ro)
    wait_out(198, 0)
    wait_out(199, 1)


_lookup = functools.partial(
    pl.kernel,
    out_type=jax.ShapeDtypeStruct((SEQ, D_MODEL, BATCH), jnp.float32),
    mesh=plsc.VectorSubcoreMesh(core_axis_name="c", subcore_axis_name="s"),
    compiler_params=pltpu.CompilerParams(use_tc_tiling_on_sc=True, needs_layout_passes=False),
    scratch_types=(
        [pltpu.VMEM((SEQ, BLK), jnp.int32) for _ in range(2)]
        + [pltpu.VMEM((BLK, BLK), jnp.float32) for _ in range(NBUF2)]
        + [pltpu.VMEM((D_MODEL, BLK), jnp.float32) for _ in range(2)]
        + [pltpu.SemaphoreType.DMA for _ in range(NBUF2 + 2)]
    ),
)(_lookup_body)


@jax.jit
def kernel(x, table):
    dense = _relayout(table.T)
    out = _lookup(x.T, dense)
    return out.transpose(2, 0, 1)


# speedup vs baseline: 1.9388x; 1.2876x over previous
import functools

import jax
import jax.numpy as jnp
from jax import lax
from jax.experimental import pallas as pl
from jax.experimental.pallas import tpu as pltpu
from jax.experimental.pallas import tpu_sc as plsc

D_MODEL = 64
SCALE = 8.0
NC, NS = 2, 16
NW = NC * NS
VOCAB = 1000000
PACK = VOCAB // 2
BATCH = 4096
SEQ = 200
LANES = 16
BLK = 128
NBUF = 3


def _iota16():
    return lax.iota(jnp.int32, LANES)


def _lookup_body(xt_hbm, dense_hbm, out_hbm, idx_v, hall,
                 gb0, gb1, gb2, ob0, ob1, sg0, sg1, sg2, so0, so1):
    c = lax.axis_index("c")
    s = lax.axis_index("s")
    wid = s * NC + c
    i0 = wid * BLK
    gbs = (gb0, gb1, gb2)
    obs = (ob0, ob1)
    sgs, sos = (sg0, sg1, sg2), (so0, so1)
    it16 = _iota16()
    kvecs = [it16 + 16 * g for g in range(8)]

    pltpu.sync_copy(xt_hbm.at[:, pl.ds(i0, BLK)], idx_v)

    @plsc.parallel_loop(0, SEQ, unroll=2)
    def _prep(t):
        for g in range(8):
            sl = pl.ds(16 * g, LANES)
            v = idx_v[t, sl]
            hall[t, sl] = (v & 1) * D_MODEL
            idx_v[t, sl] = lax.shift_right_logical(v, 1)

    def issue_gather(t, r):
        pltpu.async_copy(dense_hbm.at[idx_v.at[t]], gbs[r], sgs[r])

    def wait_gather(t, r):
        pltpu.make_async_copy(dense_hbm.at[idx_v.at[t]], gbs[r], sgs[r]).wait()

    def out_dst(t):
        return out_hbm.at[t, :, wid]

    def issue_out(t, r):
        pltpu.async_copy(obs[r], out_dst(t), sos[r])

    def wait_out(t, r):
        pltpu.make_async_copy(obs[r], out_dst(t), sos[r]).wait()

    def transpose_block(t, rg, ro):
        gb, ob = gbs[rg], obs[ro]
        hvs = [hall[t, pl.ds(16 * g, LANES)] for g in range(8)]

        @plsc.parallel_loop(0, D_MODEL, unroll=2)
        def rowd(d):
            dr = d // 8
            dl = d % 8
            vs = [plsc.load_gather(gb, [kvecs[g], hvs[g] + d])
                  for g in range(8)]
            for g in range(8):
                ob[dr, dl, pl.ds(16 * g, LANES)] = vs[g] * SCALE

    for t in range(NBUF):
        issue_gather(t, t)

    def six_body(m, _):
        base = 6 * m
        for j in range(6):
            t = base + j
            rg = j % NBUF
            ro = j % 2
            wait_gather(t, rg)

            @pl.when(t >= 2)
            def _():
                wait_out(t - 2, ro)

            transpose_block(t, rg, ro)
            issue_out(t, ro)

            @pl.when(t + NBUF < SEQ)
            def _():
                issue_gather(t + NBUF, rg)
        return 0

    lax.fori_loop(0, 33, six_body, 0)
    for t in (198, 199):
        rg = t % NBUF
        ro = t % 2
        wait_gather(t, rg)
        wait_out(t - 2, ro)
        transpose_block(t, rg, ro)
        issue_out(t, ro)
    wait_out(198, 0)
    wait_out(199, 1)


_lookup = functools.partial(
    pl.kernel,
    out_type=jax.ShapeDtypeStruct((SEQ, 8, NW, 8, BLK), jnp.float32),
    mesh=plsc.VectorSubcoreMesh(core_axis_name="c", subcore_axis_name="s"),
    compiler_params=pltpu.CompilerParams(use_tc_tiling_on_sc=False,
                                         needs_layout_passes=False),
    scratch_types=(
        [pltpu.VMEM((SEQ, BLK), jnp.int32) for _ in range(2)]
        + [pltpu.VMEM((BLK, BLK), jnp.float32) for _ in range(NBUF)]
        + [pltpu.VMEM((8, 8, BLK), jnp.float32) for _ in range(2)]
        + [pltpu.SemaphoreType.DMA for _ in range(NBUF + 2)]
    ),
)(_lookup_body)


@jax.jit
def kernel(x, table):
    dense = table.reshape(PACK, 2 * D_MODEL)
    out5 = _lookup(x.T, dense)
    out = out5.transpose(0, 1, 3, 2, 4).reshape(SEQ, D_MODEL, BATCH)
    return out.transpose(2, 0, 1)
